# baseline (speedup 1.0000x reference)
import functools

import jax
import jax.numpy as jnp
from jax import lax
from jax.experimental import pallas as pl
from jax.experimental.pallas import tpu as pltpu
from jax.experimental.pallas import tpu_sc as plsc

BATCH = 4096
NUM_PROMPTS = 4096
DIM = 2048
K = 8


BB = 512
PB = 512


def _topk_body(x_ref, k_ref, out_ref, sims_ref):
    j = pl.program_id(1)
    sims_ref[:, pl.ds(j * PB, PB)] = lax.dot_general(
        x_ref[...], k_ref[...],
        dimension_numbers=(((1,), (1,)), ((), ())),
        preferred_element_type=jnp.float32,
    )

    @pl.when(j == pl.num_programs(1) - 1)
    def _():
        iota = lax.broadcasted_iota(jnp.int32, (BB, NUM_PROMPTS), 1)
        cols = []
        for t in range(K):
            m = jnp.max(sims_ref[...], axis=1, keepdims=True)
            idx = jnp.min(jnp.where(sims_ref[...] == m, iota, NUM_PROMPTS),
                          axis=1).astype(jnp.int32).reshape(BB, 1)
            cols.append(idx)
            if t < K - 1:
                sims_ref[...] = jnp.where(iota == idx, -jnp.inf,
                                          sims_ref[...])
        out_ref[...] = jnp.concatenate(cols, axis=1)


def _tc_topk(x, keys):
    batch = x.shape[0]
    return pl.pallas_call(
        _topk_body,
        grid=(batch // BB, NUM_PROMPTS // PB),
        in_specs=[
            pl.BlockSpec((BB, DIM), lambda i, j: (i, 0)),
            pl.BlockSpec((PB, DIM), lambda i, j: (j, 0)),
        ],
        out_specs=pl.BlockSpec((BB, K), lambda i, j: (i, 0)),
        out_shape=jax.ShapeDtypeStruct((batch, K), jnp.int32),
        scratch_shapes=[pltpu.VMEM((BB, NUM_PROMPTS), jnp.float32)],
        compiler_params=pltpu.CompilerParams(
            dimension_semantics=("parallel", "arbitrary")),
    )(x, keys)


NC, NS = 2, 16
NW = NC * NS
B_TOTAL = BATCH * K
S = 4
SLICE_B = BATCH // S
SLICE_ROWS = SLICE_B * K
R_PER_W = SLICE_ROWS // NW
CB = 16
NBUF = 2
N_CHUNKS = R_PER_W // CB


def _sc_gather_slice(values, flat_idx, out_ref, t):
    mesh = plsc.VectorSubcoreMesh(core_axis_name="c", subcore_axis_name="s")

    @functools.partial(
        pl.kernel, mesh=mesh,
        out_type=(),
        scratch_types=[
            pltpu.VMEM((R_PER_W,), jnp.int32),
            pltpu.VMEM((NBUF, CB, DIM), jnp.float32),
            pltpu.SemaphoreType.DMA,
            pltpu.SemaphoreType.DMA,
        ],
    )
    def gather_kernel(values_hbm, idx_hbm, out_hbm, idx_all, rows_v, s0, s1):
        wid = lax.axis_index("s") * NC + lax.axis_index("c")
        base0 = wid * R_PER_W
        sems = [s0, s1]
        pltpu.sync_copy(idx_hbm.at[pl.ds(base0, R_PER_W)], idx_all)

        handles = [None] * NBUF

        def start(g):
            b = g % NBUF
            handles[b] = pltpu.async_copy(
                values_hbm.at[idx_all.at[pl.ds(g * CB, CB)]],
                rows_v.at[b], sems[b])

        start(0)
        out_base = t * SLICE_ROWS + base0
        for g in range(N_CHUNKS):
            if g + 1 < N_CHUNKS:
                start(g + 1)
            b = g % NBUF
            handles[b].wait()
            pltpu.sync_copy(rows_v.at[b],
                            out_hbm.at[pl.ds(out_base + g * CB, CB)])

    gather_kernel(values, flat_idx, out_ref)


def kernel(input_embedding, prompt_keys, prompt_values, top_k):
    out_ref = jax.new_ref(jnp.zeros((B_TOTAL, DIM), jnp.float32))
    for t in range(S):
        x_t = input_embedding[t * SLICE_B:(t + 1) * SLICE_B]
        idx = _tc_topk(x_t, prompt_keys) + (top_k - K)
        _sc_gather_slice(prompt_values, idx.reshape(-1), out_ref, t)
    return out_ref[...].reshape(BATCH, K, DIM)


# speedup vs baseline: 1.2767x; 1.2767x over previous
import functools

import jax
import jax.numpy as jnp
from jax import lax
from jax.experimental import pallas as pl
from jax.experimental.pallas import tpu as pltpu
from jax.experimental.pallas import tpu_sc as plsc

BATCH = 4096
NUM_PROMPTS = 4096
DIM = 2048
K = 8


BB = 512
PB = 512


def _topk_body(x_ref, k_ref, out_ref, sims_ref):
    j = pl.program_id(1)
    sims_ref[:, pl.ds(j * PB, PB)] = lax.dot_general(
        x_ref[...], k_ref[...],
        dimension_numbers=(((1,), (1,)), ((), ())),
        preferred_element_type=jnp.float32,
    )

    @pl.when(j == pl.num_programs(1) - 1)
    def _():
        iota = lax.broadcasted_iota(jnp.int32, (BB, NUM_PROMPTS), 1)
        cols = []
        for t in range(K):
            m = jnp.max(sims_ref[...], axis=1, keepdims=True)
            idx = jnp.min(jnp.where(sims_ref[...] == m, iota, NUM_PROMPTS),
                          axis=1).astype(jnp.int32).reshape(BB, 1)
            cols.append(idx)
            if t < K - 1:
                sims_ref[...] = jnp.where(iota == idx, -jnp.inf,
                                          sims_ref[...])
        out_ref[...] = jnp.concatenate(cols, axis=1)


def _tc_topk(x, keys):
    batch = x.shape[0]
    return pl.pallas_call(
        _topk_body,
        grid=(batch // BB, NUM_PROMPTS // PB),
        in_specs=[
            pl.BlockSpec((BB, DIM), lambda i, j: (i, 0)),
            pl.BlockSpec((PB, DIM), lambda i, j: (j, 0)),
        ],
        out_specs=pl.BlockSpec((BB, K), lambda i, j: (i, 0)),
        out_shape=jax.ShapeDtypeStruct((batch, K), jnp.int32),
        scratch_shapes=[pltpu.VMEM((BB, NUM_PROMPTS), jnp.float32)],
        compiler_params=pltpu.CompilerParams(
            dimension_semantics=("parallel", "arbitrary")),
    )(x, keys)


NC, NS = 2, 16
NW = NC * NS
B_TOTAL = BATCH * K
B_PER_W = B_TOTAL // NW
CB = 16
NBUF = 3
N_CHUNKS = B_PER_W // CB


def _sc_gather(values, flat_idx):
    mesh = plsc.VectorSubcoreMesh(core_axis_name="c", subcore_axis_name="s")

    @functools.partial(
        pl.kernel, mesh=mesh,
        out_type=jax.ShapeDtypeStruct((B_TOTAL, DIM), jnp.float32),
        scratch_types=[
            pltpu.VMEM((B_PER_W,), jnp.int32),
            pltpu.VMEM((NBUF, CB, DIM), jnp.float32),
            pltpu.SemaphoreType.DMA,
            pltpu.SemaphoreType.DMA,
            pltpu.SemaphoreType.DMA,
        ],
    )
    def gather_kernel(values_hbm, idx_hbm, out_hbm, idx_all, rows_v,
                      s0, s1, s2):
        wid = lax.axis_index("s") * NC + lax.axis_index("c")
        base0 = wid * B_PER_W
        sems = [s0, s1, s2]
        pltpu.sync_copy(idx_hbm.at[pl.ds(base0, B_PER_W)], idx_all)

        handles = [None] * NBUF

        def start(g):
            b = g % NBUF
            handles[b] = pltpu.async_copy(
                values_hbm.at[idx_all.at[pl.ds(g * CB, CB)]],
                rows_v.at[b], sems[b])

        start(0)
        start(1)
        for g in range(N_CHUNKS):
            if g + 2 < N_CHUNKS:
                start(g + 2)
            b = g % NBUF
            handles[b].wait()
            pltpu.sync_copy(rows_v.at[b],
                            out_hbm.at[pl.ds(base0 + g * CB, CB)])

    return gather_kernel(values, flat_idx)


def kernel(input_embedding, prompt_keys, prompt_values, top_k):
    idx = _tc_topk(input_embedding.astype(jnp.bfloat16),
                   prompt_keys.astype(jnp.bfloat16))
    idx = idx + (top_k - K)
    out = _sc_gather(prompt_values, idx.reshape(-1))
    return out.reshape(BATCH, K, DIM)


# speedup vs baseline: 1.3425x; 1.0515x over previous
import functools

import jax
import jax.numpy as jnp
from jax import lax
from jax.experimental import pallas as pl
from jax.experimental.pallas import tpu as pltpu
from jax.experimental.pallas import tpu_sc as plsc

BATCH = 4096
NUM_PROMPTS = 4096
DIM = 2048
K = 8


BB = 512
PB = 512


def _topk_body(x_ref, k_ref, out_ref, sims_ref):
    j = pl.program_id(1)
    sims_ref[:, pl.ds(j * PB, PB)] = lax.dot_general(
        x_ref[...].astype(jnp.bfloat16), k_ref[...],
        dimension_numbers=(((1,), (1,)), ((), ())),
        preferred_element_type=jnp.float32,
    )

    @pl.when(j == pl.num_programs(1) - 1)
    def _():
        iota = lax.broadcasted_iota(jnp.int32, (BB, NUM_PROMPTS), 1)
        cols = []
        s = sims_ref[...]
        m = jnp.max(s, axis=1, keepdims=True)
        for t in range(K):
            idx = jnp.min(jnp.where(s == m, iota, NUM_PROMPTS),
                          axis=1).astype(jnp.int32).reshape(BB, 1)
            cols.append(idx)
            if t < K - 1:
                s = jnp.where(iota == idx, -jnp.inf, s)
                m = jnp.max(s, axis=1, keepdims=True)
        out_ref[...] = jnp.concatenate(cols, axis=1)


def _tc_topk(x, keys):
    batch = x.shape[0]
    return pl.pallas_call(
        _topk_body,
        grid=(batch // BB, NUM_PROMPTS // PB),
        in_specs=[
            pl.BlockSpec((BB, DIM), lambda i, j: (i, 0)),
            pl.BlockSpec((PB, DIM), lambda i, j: (j, 0)),
        ],
        out_specs=pl.BlockSpec((BB, K), lambda i, j: (i, 0)),
        out_shape=jax.ShapeDtypeStruct((batch, K), jnp.int32),
        scratch_shapes=[pltpu.VMEM((BB, NUM_PROMPTS), jnp.float32)],
        compiler_params=pltpu.CompilerParams(
            dimension_semantics=("parallel", "arbitrary")),
    )(x, keys)


NC, NS = 2, 16
NW = NC * NS
B_TOTAL = BATCH * K
B_PER_W = B_TOTAL // NW
CB = 8
NBUF = 6
N_CHUNKS = B_PER_W // CB


def _sc_gather(values, flat_idx):
    mesh = plsc.VectorSubcoreMesh(core_axis_name="c", subcore_axis_name="s")

    @functools.partial(
        pl.kernel, mesh=mesh,
        out_type=jax.ShapeDtypeStruct((B_TOTAL, DIM), jnp.float32),
        scratch_types=[
            pltpu.VMEM((B_PER_W,), jnp.int32),
            pltpu.VMEM((NBUF, CB, DIM), jnp.float32),
            [pltpu.SemaphoreType.DMA] * NBUF,
            [pltpu.SemaphoreType.DMA] * NBUF,
        ],
    )
    def gather_kernel(values_hbm, idx_hbm, out_hbm, idx_all, rows_v,
                      gsems, ssems):
        wid = lax.axis_index("s") * NC + lax.axis_index("c")
        base0 = wid * B_PER_W
        pltpu.sync_copy(idx_hbm.at[pl.ds(base0, B_PER_W)], idx_all)

        gh = [None] * NBUF
        sh = [None] * NBUF

        def start_gather(g):
            b = g % NBUF
            gh[b] = pltpu.async_copy(
                values_hbm.at[idx_all.at[pl.ds(g * CB, CB)]],
                rows_v.at[b], gsems[b])

        for g in range(NBUF):
            start_gather(g)
        for g in range(N_CHUNKS):
            b = g % NBUF
            gh[b].wait()
            sh[b] = pltpu.async_copy(
                rows_v.at[b], out_hbm.at[pl.ds(base0 + g * CB, CB)],
                ssems[b])
            ng = g + NBUF
            if ng < N_CHUNKS:
                sh[b].wait()
                start_gather(ng)
        for b in range(min(NBUF, N_CHUNKS)):
            last = N_CHUNKS - 1 - b
            if last >= max(0, N_CHUNKS - NBUF):
                sh[last % NBUF].wait()

    return gather_kernel(values, flat_idx)


def kernel(input_embedding, prompt_keys, prompt_values, top_k):
    idx = _tc_topk(input_embedding, prompt_keys.astype(jnp.bfloat16))
    idx = idx + (top_k - K)
    out = _sc_gather(prompt_values, idx.reshape(-1))
    return out.reshape(BATCH, K, DIM)


# speedup vs baseline: 1.3858x; 1.0322x over previous
import functools

import jax
import jax.numpy as jnp
from jax import lax
from jax.experimental import pallas as pl
from jax.experimental.pallas import tpu as pltpu
from jax.experimental.pallas import tpu_sc as plsc

BATCH = 4096
NUM_PROMPTS = 4096
DIM = 2048
K = 8


BB = 512
PB = 1024


def _topk_body(x_ref, k_ref, out_ref, sims_ref):
    j = pl.program_id(1)
    sims_ref[:, pl.ds(j * PB, PB)] = lax.dot_general(
        x_ref[...].astype(jnp.bfloat16), k_ref[...],
        dimension_numbers=(((1,), (1,)), ((), ())),
        preferred_element_type=jnp.float32,
    )

    @pl.when(j == pl.num_programs(1) - 1)
    def _():
        iota = lax.broadcasted_iota(jnp.int32, (BB, NUM_PROMPTS), 1)
        cols = []
        s = sims_ref[...]
        m = jnp.max(s, axis=1, keepdims=True)
        for t in range(K):
            idx = jnp.min(jnp.where(s == m, iota, NUM_PROMPTS),
                          axis=1).astype(jnp.int32).reshape(BB, 1)
            cols.append(idx)
            if t < K - 1:
                s = jnp.where(iota == idx, -jnp.inf, s)
                m = jnp.max(s, axis=1, keepdims=True)
        out_ref[...] = jnp.concatenate(cols, axis=1)


def _tc_topk(x, keys):
    batch = x.shape[0]
    return pl.pallas_call(
        _topk_body,
        grid=(batch // BB, NUM_PROMPTS // PB),
        in_specs=[
            pl.BlockSpec((BB, DIM), lambda i, j: (i, 0)),
            pl.BlockSpec((PB, DIM), lambda i, j: (j, 0)),
        ],
        out_specs=pl.BlockSpec((BB, K), lambda i, j: (i, 0)),
        out_shape=jax.ShapeDtypeStruct((batch, K), jnp.int32),
        scratch_shapes=[pltpu.VMEM((BB, NUM_PROMPTS), jnp.float32)],
        compiler_params=pltpu.CompilerParams(
            dimension_semantics=("parallel", "arbitrary")),
    )(x, keys)


NC, NS = 2, 16
NW = NC * NS
B_TOTAL = BATCH * K
B_PER_W = B_TOTAL // NW
CB = 8
NBUF = 6
N_CHUNKS = B_PER_W // CB


def _sc_gather(values, flat_idx):
    mesh = plsc.VectorSubcoreMesh(core_axis_name="c", subcore_axis_name="s")

    @functools.partial(
        pl.kernel, mesh=mesh,
        out_type=jax.ShapeDtypeStruct((B_TOTAL, DIM), jnp.float32),
        scratch_types=[
            pltpu.VMEM((B_PER_W,), jnp.int32),
            pltpu.VMEM((NBUF, CB, DIM), jnp.float32),
            [pltpu.SemaphoreType.DMA] * NBUF,
            [pltpu.SemaphoreType.DMA] * NBUF,
        ],
    )
    def gather_kernel(values_hbm, idx_hbm, out_hbm, idx_all, rows_v,
                      gsems, ssems):
        wid = lax.axis_index("s") * NC + lax.axis_index("c")
        base0 = wid * B_PER_W
        pltpu.sync_copy(idx_hbm.at[pl.ds(base0, B_PER_W)], idx_all)

        gh = [None] * NBUF
        sh = [None] * NBUF

        def start_gather(g):
            b = g % NBUF
            gh[b] = pltpu.async_copy(
                values_hbm.at[idx_all.at[pl.ds(g * CB, CB)]],
                rows_v.at[b], gsems[b])

        for g in range(NBUF):
            start_gather(g)
        for g in range(N_CHUNKS):
            b = g % NBUF
            gh[b].wait()
            sh[b] = pltpu.async_copy(
                rows_v.at[b], out_hbm.at[pl.ds(base0 + g * CB, CB)],
                ssems[b])
            ng = g + NBUF
            if ng < N_CHUNKS:
                sh[b].wait()
                start_gather(ng)
        for b in range(min(NBUF, N_CHUNKS)):
            last = N_CHUNKS - 1 - b
            if last >= max(0, N_CHUNKS - NBUF):
                sh[last % NBUF].wait()

    return gather_kernel(values, flat_idx)


def kernel(input_embedding, prompt_keys, prompt_values, top_k):
    idx = _tc_topk(input_embedding, prompt_keys.astype(jnp.bfloat16))
    idx = idx + (top_k - K)
    out = _sc_gather(prompt_values, idx.reshape(-1))
    return out.reshape(BATCH, K, DIM)
